# baseline (speedup 1.0000x reference)
import jax
import jax.numpy as jnp
from jax.experimental import pallas as pl
from jax.experimental.pallas import tpu as pltpu

_B, _D, _N = 16, 256, 8192
_K = 2048


def _norm_body(se_ref, te_ref, sn_ref, tn_ref):
    se = se_ref[0]
    te = te_ref[0]
    sn_ref[0, 0, :] = jnp.sqrt(jnp.sum(se * se, axis=0))
    tn_ref[0, 0, :] = jnp.sqrt(jnp.sum(te * te, axis=0))


def _norms(src_embedding, tgt_embedding):
    return pl.pallas_call(
        _norm_body,
        grid=(_B,),
        in_specs=[
            pl.BlockSpec((1, _D, _N), lambda b: (b, 0, 0)),
            pl.BlockSpec((1, _D, _N), lambda b: (b, 0, 0)),
        ],
        out_specs=[
            pl.BlockSpec((1, 1, _N), lambda b: (b, 0, 0)),
            pl.BlockSpec((1, 1, _N), lambda b: (b, 0, 0)),
        ],
        out_shape=[
            jax.ShapeDtypeStruct((_B, 1, _N), jnp.float32),
            jax.ShapeDtypeStruct((_B, 1, _N), jnp.float32),
        ],
    )(src_embedding, tgt_embedding)


def kernel(src, tgt, src_embedding, tgt_embedding):
    src_norm, tgt_norm = _norms(src_embedding, tgt_embedding)
    src_norm = src_norm[:, 0, :]
    tgt_norm = tgt_norm[:, 0, :]
    _, src_idx = jax.lax.top_k(src_norm, _K)
    _, tgt_idx = jax.lax.top_k(tgt_norm, _K)
    src_kp = jnp.take_along_axis(src, src_idx[:, None, :], axis=2)
    tgt_kp = jnp.take_along_axis(tgt, tgt_idx[:, None, :], axis=2)
    src_emb_kp = jnp.take_along_axis(src_embedding, src_idx[:, None, :], axis=2)
    tgt_emb_kp = jnp.take_along_axis(tgt_embedding, tgt_idx[:, None, :], axis=2)
    return (src_kp, tgt_kp, src_emb_kp, tgt_emb_kp)
